# baseline (speedup 1.0000x reference)
import numpy as np
import jax
import jax.numpy as jnp
from jax import lax
from jax.experimental import pallas as pl
from jax.experimental.pallas import tpu as pltpu
from jax.experimental.pallas import tpu_sc as plsc

_N = 8388608
_NC, _NS = 2, 16
_NW = _NC * _NS
_PER_W = _N // _NW
_C = 16384
_S1 = _PER_W // _C
_NSTEP = _N // _C
_T = 16384
_D = _N // _T
_BPW = _D // _NW
_L = 16

_U32 = np.uint32


def _threefry2x32(k1, k2, x1, x2):
    rot = ((13, 15, 26, 6), (17, 29, 16, 24))
    ks = (k1, k2, _U32(k1 ^ k2 ^ _U32(0x1BD11BDA)))
    x = [(x1 + ks[0]).astype(_U32), (x2 + ks[1]).astype(_U32)]
    for i in range(1, 6):
        for d in rot[(i - 1) % 2]:
            x[0] = (x[0] + x[1]).astype(_U32)
            x[1] = ((x[1] << _U32(d)) | (x[1] >> _U32(32 - d))).astype(_U32)
            x[1] = x[0] ^ x[1]
        x[0] = (x[0] + ks[i % 3]).astype(_U32)
        x[1] = (x[1] + ks[(i + 1) % 3] + _U32(i)).astype(_U32)
    return x


def _fixed_perm(seed, n):
    key = (_U32(0), _U32(seed))
    x = np.arange(n, dtype=np.int64)
    num_rounds = int(np.ceil(3 * np.log(n) / np.log(np.iinfo(np.uint32).max)))
    for _ in range(num_rounds):
        hi, lo = np.zeros(2, _U32), np.arange(2, dtype=_U32)
        b1, b2 = _threefry2x32(key[0], key[1], hi, lo)
        key, subkey = (b1[0], b2[0]), (b1[1], b2[1])
        chi = np.zeros(n, _U32)
        clo = np.arange(n, dtype=np.uint64).astype(_U32)
        s1, s2 = _threefry2x32(subkey[0], subkey[1], chi, clo)
        x = x[np.argsort(s1 ^ s2, kind="stable")]
    return x


_sched_cache = []


def _schedule():
    if _sched_cache:
        return _sched_cache[0]
    perm = _fixed_perm(42, _N)
    inv = np.empty(_N, np.int64)
    inv[perm] = np.arange(_N)
    bj = inv // _T

    zpos = np.empty(_N, np.int64)
    l1 = np.empty(_N, np.int32)
    for g in range(_NSTEP):
        sl = slice(g * _C, (g + 1) * _C)
        order = np.argsort(bj[sl], kind="stable").astype(np.int32)
        l1[sl] = order
        zpos[g * _C + order] = g * _C + np.arange(_C)

    p = zpos[perm]
    prow = p // _L
    plane = (p % _L).astype(np.int32)
    rows_list = []
    l2 = np.empty(_N, np.int32)
    for b in range(_D):
        sl = slice(b * _T, (b + 1) * _T)
        rows = np.unique(prow[sl])
        rows_list.append(rows)
        l2[sl] = np.searchsorted(rows, prow[sl]).astype(np.int32) * _L + plane[sl]
    rmax = max(len(r) for r in rows_list)
    r_pad = -(-rmax // 8) * 8
    rl = np.zeros((_D, r_pad), np.int32)
    for b, rows in enumerate(rows_list):
        rl[b, : len(rows)] = rows
    _sched_cache.append((l1, rl.reshape(-1), l2, r_pad))
    return _sched_cache[0]


def _p1_body(x_hbm, l1_hbm, z_hbm, src0, src1, li0, li1, st0, st1,
             xs0, xs1, ls0, ls1, os0, os1):
    wid = lax.axis_index("s") * _NC + lax.axis_index("c")
    base = wid * _PER_W
    src = (src0, src1)
    li = (li0, li1)
    st = (st0, st1)
    xs = (xs0, xs1)
    ls = (ls0, ls1)
    osm = (os0, os1)

    def in_copies(s):
        off = base + s * _C
        return (
            pltpu.async_copy(x_hbm.at[pl.ds(off, _C)], src[s % 2], xs[s % 2]),
            pltpu.async_copy(l1_hbm.at[pl.ds(off, _C)], li[s % 2], ls[s % 2]),
        )

    ics = {0: in_copies(0)}
    oc = {}
    for s in range(_S1):
        if s + 1 < _S1:
            ics[s + 1] = in_copies(s + 1)
        for d in ics.pop(s):
            d.wait()
        if s >= 2:
            oc[s - 2].wait()
        cur = s % 2
        src_r, li_r, st_r = src[cur], li[cur], st[cur]

        @plsc.parallel_loop(0, _C // _L, unroll=8)
        def _(k, src_r=src_r, li_r=li_r, st_r=st_r):
            idx16 = li_r[pl.ds(k * _L, _L)]
            st_r[pl.ds(k * _L, _L)] = plsc.load_gather(src_r, [idx16])

        oc[s] = pltpu.async_copy(
            st_r, z_hbm.at[pl.ds(base + s * _C, _C)], osm[cur])
    oc[_S1 - 2].wait()
    oc[_S1 - 1].wait()


def kernel(x):
    l1_np, rl_np, l2_np, r_pad = _schedule()
    mesh = plsc.VectorSubcoreMesh(core_axis_name="c", subcore_axis_name="s")

    cparams = pltpu.CompilerParams(
        needs_layout_passes=False, use_tc_tiling_on_sc=False)
    p1 = pl.kernel(
        _p1_body,
        out_type=jax.ShapeDtypeStruct((_N,), jnp.float32),
        mesh=mesh,
        compiler_params=cparams,
        scratch_types=[
            pltpu.VMEM((_C,), jnp.float32),
            pltpu.VMEM((_C,), jnp.float32),
            pltpu.VMEM((_C,), jnp.int32),
            pltpu.VMEM((_C,), jnp.int32),
            pltpu.VMEM((_C,), jnp.float32),
            pltpu.VMEM((_C,), jnp.float32),
        ] + [pltpu.SemaphoreType.DMA] * 6,
    )

    def _p2_body(z2_hbm, rl_hbm, l2_hbm, out_hbm, rl0, rl1, rw0, rw1,
                 li0, li1, ob0, ob1, rs0, rs1, gs0, gs1, ls0, ls1, os0, os1):
        wid = lax.axis_index("s") * _NC + lax.axis_index("c")
        bbase = wid * _BPW
        rl = (rl0, rl1)
        rw = (rw0, rw1)
        li = (li0, li1)
        ob = (ob0, ob1)
        rs = (rs0, rs1)
        gs = (gs0, gs1)
        ls = (ls0, ls1)
        osm = (os0, os1)

        def rl_copy(s):
            src = rl_hbm.at[pl.ds((bbase + s) * r_pad, r_pad)]
            return pltpu.async_copy(src, rl[s % 2], rs[s % 2])

        def l2_copy(s):
            src = l2_hbm.at[pl.ds((bbase + s) * _T, _T)]
            return pltpu.async_copy(src, li[s % 2], ls[s % 2])

        def row_gather(s):
            return pltpu.async_copy(z2_hbm.at[rl[s % 2]], rw[s % 2], gs[s % 2])

        def out_copy(s):
            dst = out_hbm.at[pl.ds((bbase + s) * _T, _T)]
            return pltpu.async_copy(ob[s % 2], dst, osm[s % 2])

        rlc = {0: rl_copy(0)}
        l2c = {0: l2_copy(0)}
        rlc[0].wait()
        rg = {0: row_gather(0)}
        rlc[1] = rl_copy(1)
        l2c[1] = l2_copy(1)
        oc = {}
        for s in range(_BPW):
            if s + 1 < _BPW:
                rlc[s + 1].wait()
                rg[s + 1] = row_gather(s + 1)
            rg[s].wait()
            l2c[s].wait()
            if s >= 2:
                oc[s - 2].wait()
            cur = s % 2
            rw_r, li_r, ob_r = rw[cur], li[cur], ob[cur]

            @plsc.parallel_loop(0, _T // _L, unroll=8)
            def _(k, rw_r=rw_r, li_r=li_r, ob_r=ob_r):
                idx16 = li_r[pl.ds(k * _L, _L)]
                r16 = lax.shift_right_logical(idx16, 4)
                c16 = lax.bitwise_and(idx16, 15)
                ob_r[pl.ds(k * _L, _L)] = plsc.load_gather(rw_r, [r16, c16])

            oc[s] = out_copy(s)
            if s + 2 < _BPW:
                rlc[s + 2] = rl_copy(s + 2)
                l2c[s + 2] = l2_copy(s + 2)
        oc[_BPW - 2].wait()
        oc[_BPW - 1].wait()

    p2 = pl.kernel(
        _p2_body,
        out_type=jax.ShapeDtypeStruct((_N,), jnp.float32),
        mesh=mesh,
        compiler_params=cparams,
        scratch_types=[
            pltpu.VMEM((r_pad,), jnp.int32),
            pltpu.VMEM((r_pad,), jnp.int32),
            pltpu.VMEM((r_pad, _L), jnp.float32),
            pltpu.VMEM((r_pad, _L), jnp.float32),
            pltpu.VMEM((_T,), jnp.int32),
            pltpu.VMEM((_T,), jnp.int32),
            pltpu.VMEM((_T,), jnp.float32),
            pltpu.VMEM((_T,), jnp.float32),
        ] + [pltpu.SemaphoreType.DMA] * 8,
    )

    z = p1(x, jnp.asarray(l1_np))
    out = p2(z.reshape(_N // _L, _L), jnp.asarray(rl_np), jnp.asarray(l2_np))
    return out


# speedup vs baseline: 1.0011x; 1.0011x over previous
import numpy as np
import jax
import jax.numpy as jnp
from jax import lax
from jax.experimental import pallas as pl
from jax.experimental.pallas import tpu as pltpu
from jax.experimental.pallas import tpu_sc as plsc

_N = 8388608
_NC, _NS = 2, 16
_NW = _NC * _NS
_PER_W = _N // _NW
_C = 16384
_S1 = _PER_W // _C
_NSTEP = _N // _C
_T = 16384
_D = _N // _T
_BPW = _D // _NW
_L = 16

_U32 = np.uint32


def _threefry2x32(k1, k2, x1, x2):
    rot = ((13, 15, 26, 6), (17, 29, 16, 24))
    ks = (k1, k2, _U32(k1 ^ k2 ^ _U32(0x1BD11BDA)))
    x = [(x1 + ks[0]).astype(_U32), (x2 + ks[1]).astype(_U32)]
    for i in range(1, 6):
        for d in rot[(i - 1) % 2]:
            x[0] = (x[0] + x[1]).astype(_U32)
            x[1] = ((x[1] << _U32(d)) | (x[1] >> _U32(32 - d))).astype(_U32)
            x[1] = x[0] ^ x[1]
        x[0] = (x[0] + ks[i % 3]).astype(_U32)
        x[1] = (x[1] + ks[(i + 1) % 3] + _U32(i)).astype(_U32)
    return x


def _fixed_perm(seed, n):
    key = (_U32(0), _U32(seed))
    x = np.arange(n, dtype=np.int64)
    num_rounds = int(np.ceil(3 * np.log(n) / np.log(np.iinfo(np.uint32).max)))
    for _ in range(num_rounds):
        hi, lo = np.zeros(2, _U32), np.arange(2, dtype=_U32)
        b1, b2 = _threefry2x32(key[0], key[1], hi, lo)
        key, subkey = (b1[0], b2[0]), (b1[1], b2[1])
        chi = np.zeros(n, _U32)
        clo = np.arange(n, dtype=np.uint64).astype(_U32)
        s1, s2 = _threefry2x32(subkey[0], subkey[1], chi, clo)
        x = x[np.argsort(s1 ^ s2, kind="stable")]
    return x


_sched_cache = []


def _schedule():
    if _sched_cache:
        return _sched_cache[0]
    perm = _fixed_perm(42, _N)
    inv = np.empty(_N, np.int64)
    inv[perm] = np.arange(_N)
    bj = inv // _T

    zpos = np.empty(_N, np.int64)
    l1 = np.empty(_N, np.int32)
    for g in range(_NSTEP):
        sl = slice(g * _C, (g + 1) * _C)
        order = np.argsort(bj[sl], kind="stable").astype(np.int32)
        l1[sl] = order
        zpos[g * _C + order] = g * _C + np.arange(_C)

    p = zpos[perm]
    prow = p // _L
    plane = (p % _L).astype(np.int32)
    rows_list = []
    l2 = np.empty(_N, np.int32)
    for b in range(_D):
        sl = slice(b * _T, (b + 1) * _T)
        rows = np.unique(prow[sl])
        rows_list.append(rows)
        l2[sl] = np.searchsorted(rows, prow[sl]).astype(np.int32) * _L + plane[sl]
    rmax = max(len(r) for r in rows_list)
    r_pad = -(-rmax // 8) * 8
    rl = np.zeros((_D, r_pad), np.int32)
    for b, rows in enumerate(rows_list):
        rl[b, : len(rows)] = rows
    _sched_cache.append((l1, rl.reshape(-1), l2, r_pad))
    return _sched_cache[0]


def _p1_body(x_hbm, l1_hbm, z_hbm, src0, src1, li0, li1, st0, st1,
             xs0, xs1, ls0, ls1, os0, os1):
    wid = lax.axis_index("s") * _NC + lax.axis_index("c")
    base = wid * _PER_W
    src = (src0, src1)
    li = (li0, li1)
    st = (st0, st1)
    xs = (xs0, xs1)
    ls = (ls0, ls1)
    osm = (os0, os1)

    def in_copies(s):
        off = base + s * _C
        return (
            pltpu.async_copy(x_hbm.at[pl.ds(off, _C)], src[s % 2], xs[s % 2]),
            pltpu.async_copy(l1_hbm.at[pl.ds(off, _C)], li[s % 2], ls[s % 2]),
        )

    ics = {0: in_copies(0)}
    oc = {}
    for s in range(_S1):
        if s + 1 < _S1:
            ics[s + 1] = in_copies(s + 1)
        for d in ics.pop(s):
            d.wait()
        if s >= 2:
            oc[s - 2].wait()
        cur = s % 2
        src_r, li_r, st_r = src[cur], li[cur], st[cur]

        @plsc.parallel_loop(0, _C // _L, unroll=8)
        def _(k, src_r=src_r, li_r=li_r, st_r=st_r):
            idx16 = li_r[pl.ds(k * _L, _L)]
            st_r[k] = plsc.load_gather(src_r, [idx16])

        oc[s] = pltpu.async_copy(
            st_r, z_hbm.at[pl.ds((base + s * _C) // _L, _C // _L)], osm[cur])
    oc[_S1 - 2].wait()
    oc[_S1 - 1].wait()


def kernel(x):
    l1_np, rl_np, l2_np, r_pad = _schedule()
    mesh = plsc.VectorSubcoreMesh(core_axis_name="c", subcore_axis_name="s")

    cparams = pltpu.CompilerParams(
        needs_layout_passes=False, use_tc_tiling_on_sc=False)
    p1 = pl.kernel(
        _p1_body,
        out_type=jax.ShapeDtypeStruct((_N // _L, _L), jnp.float32),
        mesh=mesh,
        compiler_params=cparams,
        scratch_types=[
            pltpu.VMEM((_C,), jnp.float32),
            pltpu.VMEM((_C,), jnp.float32),
            pltpu.VMEM((_C,), jnp.int32),
            pltpu.VMEM((_C,), jnp.int32),
            pltpu.VMEM((_C // _L, _L), jnp.float32),
            pltpu.VMEM((_C // _L, _L), jnp.float32),
        ] + [pltpu.SemaphoreType.DMA] * 6,
    )

    def _p2_body(z2_hbm, rl_hbm, l2_hbm, out_hbm, rl0, rl1, rw0, rw1,
                 li0, li1, ob0, ob1, rs0, rs1, gs0, gs1, ls0, ls1, os0, os1):
        wid = lax.axis_index("s") * _NC + lax.axis_index("c")
        bbase = wid * _BPW
        rl = (rl0, rl1)
        rw = (rw0, rw1)
        li = (li0, li1)
        ob = (ob0, ob1)
        rs = (rs0, rs1)
        gs = (gs0, gs1)
        ls = (ls0, ls1)
        osm = (os0, os1)

        def rl_copy(s):
            src = rl_hbm.at[pl.ds((bbase + s) * r_pad, r_pad)]
            return pltpu.async_copy(src, rl[s % 2], rs[s % 2])

        def l2_copy(s):
            src = l2_hbm.at[pl.ds((bbase + s) * _T, _T)]
            return pltpu.async_copy(src, li[s % 2], ls[s % 2])

        def row_gather(s):
            return pltpu.async_copy(z2_hbm.at[rl[s % 2]], rw[s % 2], gs[s % 2])

        def out_copy(s):
            dst = out_hbm.at[pl.ds((bbase + s) * _T, _T)]
            return pltpu.async_copy(ob[s % 2], dst, osm[s % 2])

        rlc = {0: rl_copy(0)}
        l2c = {0: l2_copy(0)}
        rlc[0].wait()
        rg = {0: row_gather(0)}
        rlc[1] = rl_copy(1)
        l2c[1] = l2_copy(1)
        oc = {}
        for s in range(_BPW):
            if s + 1 < _BPW:
                rlc[s + 1].wait()
                rg[s + 1] = row_gather(s + 1)
            rg[s].wait()
            l2c[s].wait()
            if s >= 2:
                oc[s - 2].wait()
            cur = s % 2
            rw_r, li_r, ob_r = rw[cur], li[cur], ob[cur]

            @plsc.parallel_loop(0, _T // _L, unroll=8)
            def _(k, rw_r=rw_r, li_r=li_r, ob_r=ob_r):
                idx16 = li_r[pl.ds(k * _L, _L)]
                r16 = lax.shift_right_logical(idx16, 4)
                c16 = lax.bitwise_and(idx16, 15)
                ob_r[pl.ds(k * _L, _L)] = plsc.load_gather(rw_r, [r16, c16])

            oc[s] = out_copy(s)
            if s + 2 < _BPW:
                rlc[s + 2] = rl_copy(s + 2)
                l2c[s + 2] = l2_copy(s + 2)
        oc[_BPW - 2].wait()
        oc[_BPW - 1].wait()

    p2 = pl.kernel(
        _p2_body,
        out_type=jax.ShapeDtypeStruct((_N,), jnp.float32),
        mesh=mesh,
        compiler_params=cparams,
        scratch_types=[
            pltpu.VMEM((r_pad,), jnp.int32),
            pltpu.VMEM((r_pad,), jnp.int32),
            pltpu.VMEM((r_pad, _L), jnp.float32),
            pltpu.VMEM((r_pad, _L), jnp.float32),
            pltpu.VMEM((_T,), jnp.int32),
            pltpu.VMEM((_T,), jnp.int32),
            pltpu.VMEM((_T,), jnp.float32),
            pltpu.VMEM((_T,), jnp.float32),
        ] + [pltpu.SemaphoreType.DMA] * 8,
    )

    z = p1(x, jnp.asarray(l1_np))
    out = p2(z, jnp.asarray(rl_np), jnp.asarray(l2_np))
    return out


# speedup vs baseline: 1.0017x; 1.0006x over previous
import numpy as np
import jax
import jax.numpy as jnp
from jax import lax
from jax.experimental import pallas as pl
from jax.experimental.pallas import tpu as pltpu
from jax.experimental.pallas import tpu_sc as plsc

_N = 8388608
_NC, _NS = 2, 16
_NW = _NC * _NS
_PER_W = _N // _NW
_C = 16384
_S1 = _PER_W // _C
_NSTEP = _N // _C
_T = 16384
_D = _N // _T
_BPW = _D // _NW
_L = 16

_U32 = np.uint32


def _threefry2x32(k1, k2, x1, x2):
    rot = ((13, 15, 26, 6), (17, 29, 16, 24))
    ks = (k1, k2, _U32(k1 ^ k2 ^ _U32(0x1BD11BDA)))
    x = [(x1 + ks[0]).astype(_U32), (x2 + ks[1]).astype(_U32)]
    for i in range(1, 6):
        for d in rot[(i - 1) % 2]:
            x[0] = (x[0] + x[1]).astype(_U32)
            x[1] = ((x[1] << _U32(d)) | (x[1] >> _U32(32 - d))).astype(_U32)
            x[1] = x[0] ^ x[1]
        x[0] = (x[0] + ks[i % 3]).astype(_U32)
        x[1] = (x[1] + ks[(i + 1) % 3] + _U32(i)).astype(_U32)
    return x


def _fixed_perm(seed, n):
    key = (_U32(0), _U32(seed))
    x = np.arange(n, dtype=np.int64)
    num_rounds = int(np.ceil(3 * np.log(n) / np.log(np.iinfo(np.uint32).max)))
    for _ in range(num_rounds):
        hi, lo = np.zeros(2, _U32), np.arange(2, dtype=_U32)
        b1, b2 = _threefry2x32(key[0], key[1], hi, lo)
        key, subkey = (b1[0], b2[0]), (b1[1], b2[1])
        chi = np.zeros(n, _U32)
        clo = np.arange(n, dtype=np.uint64).astype(_U32)
        s1, s2 = _threefry2x32(subkey[0], subkey[1], chi, clo)
        x = x[np.argsort(s1 ^ s2, kind="stable")]
    return x


_sched_cache = []


def _schedule():
    if _sched_cache:
        return _sched_cache[0]
    perm = _fixed_perm(42, _N)
    inv = np.empty(_N, np.int64)
    inv[perm] = np.arange(_N)
    bj = inv // _T

    zpos = np.empty(_N, np.int64)
    l1 = np.empty(_N, np.int32)
    for g in range(_NSTEP):
        sl = slice(g * _C, (g + 1) * _C)
        order = np.argsort(bj[sl], kind="stable").astype(np.int32)
        l1[sl] = order
        zpos[g * _C + order] = g * _C + np.arange(_C)

    p = zpos[perm]
    prow = p // _L
    plane = (p % _L).astype(np.int32)
    rows_list = []
    l2 = np.empty(_N, np.int32)
    for b in range(_D):
        sl = slice(b * _T, (b + 1) * _T)
        rows = np.unique(prow[sl])
        rows_list.append(rows)
        l2[sl] = np.searchsorted(rows, prow[sl]).astype(np.int32) * _L + plane[sl]
    rmax = max(len(r) for r in rows_list)
    r_pad = -(-rmax // 8) * 8
    rl = np.zeros((_D, r_pad), np.int32)
    for b, rows in enumerate(rows_list):
        rl[b, : len(rows)] = rows
    _sched_cache.append((l1, rl.reshape(-1), l2, r_pad))
    return _sched_cache[0]


def _p1_body(x_hbm, l1_hbm, z_hbm, src0, src1, li0, li1, st0, st1,
             xs0, xs1, ls0, ls1, os0, os1):
    wid = lax.axis_index("s") * _NC + lax.axis_index("c")
    base = wid * _PER_W
    src = (src0, src1)
    li = (li0, li1)
    st = (st0, st1)
    xs = (xs0, xs1)
    ls = (ls0, ls1)
    osm = (os0, os1)

    def in_copies(s):
        off = base + s * _C
        return (
            pltpu.async_copy(x_hbm.at[pl.ds(off, _C)], src[s % 2], xs[s % 2]),
            pltpu.async_copy(l1_hbm.at[pl.ds(off, _C)], li[s % 2], ls[s % 2]),
        )

    ics = {0: in_copies(0)}
    oc = {}
    for s in range(_S1):
        if s + 1 < _S1:
            ics[s + 1] = in_copies(s + 1)
        for d in ics.pop(s):
            d.wait()
        if s >= 2:
            oc[s - 2].wait()
        cur = s % 2
        src_r, li_r, st_r = src[cur], li[cur], st[cur]

        @plsc.parallel_loop(0, _C // _L, unroll=8)
        def _(k, src_r=src_r, li_r=li_r, st_r=st_r):
            idx16 = li_r[pl.ds(k * _L, _L)]
            st_r[k] = plsc.load_gather(src_r, [idx16])

        oc[s] = pltpu.async_copy(
            st_r, z_hbm.at[pl.ds((base + s * _C) // _L, _C // _L)], osm[cur])
    oc[_S1 - 2].wait()
    oc[_S1 - 1].wait()


def kernel(x):
    l1_np, rl_np, l2_np, r_pad = _schedule()
    mesh = plsc.VectorSubcoreMesh(core_axis_name="c", subcore_axis_name="s")

    cparams = pltpu.CompilerParams(
        needs_layout_passes=False, use_tc_tiling_on_sc=False,
        skip_device_barrier=True)
    p1 = pl.kernel(
        _p1_body,
        out_type=jax.ShapeDtypeStruct((_N // _L, _L), jnp.float32),
        mesh=mesh,
        compiler_params=cparams,
        scratch_types=[
            pltpu.VMEM((_C,), jnp.float32),
            pltpu.VMEM((_C,), jnp.float32),
            pltpu.VMEM((_C,), jnp.int32),
            pltpu.VMEM((_C,), jnp.int32),
            pltpu.VMEM((_C // _L, _L), jnp.float32),
            pltpu.VMEM((_C // _L, _L), jnp.float32),
        ] + [pltpu.SemaphoreType.DMA] * 6,
    )

    def _p2_body(z2_hbm, rl_hbm, l2_hbm, out_hbm, rl0, rl1, rw0, rw1,
                 li0, li1, ob0, ob1, rs0, rs1, gs0, gs1, ls0, ls1, os0, os1):
        wid = lax.axis_index("s") * _NC + lax.axis_index("c")
        bbase = wid * _BPW
        rl = (rl0, rl1)
        rw = (rw0, rw1)
        li = (li0, li1)
        ob = (ob0, ob1)
        rs = (rs0, rs1)
        gs = (gs0, gs1)
        ls = (ls0, ls1)
        osm = (os0, os1)

        def rl_copy(s):
            src = rl_hbm.at[pl.ds((bbase + s) * r_pad, r_pad)]
            return pltpu.async_copy(src, rl[s % 2], rs[s % 2])

        def l2_copy(s):
            src = l2_hbm.at[pl.ds((bbase + s) * _T, _T)]
            return pltpu.async_copy(src, li[s % 2], ls[s % 2])

        def row_gather(s):
            return pltpu.async_copy(z2_hbm.at[rl[s % 2]], rw[s % 2], gs[s % 2])

        def out_copy(s):
            dst = out_hbm.at[pl.ds((bbase + s) * _T, _T)]
            return pltpu.async_copy(ob[s % 2], dst, osm[s % 2])

        rlc = {0: rl_copy(0)}
        l2c = {0: l2_copy(0)}
        rlc[0].wait()
        rg = {0: row_gather(0)}
        rlc[1] = rl_copy(1)
        l2c[1] = l2_copy(1)
        oc = {}
        for s in range(_BPW):
            if s + 1 < _BPW:
                rlc[s + 1].wait()
                rg[s + 1] = row_gather(s + 1)
            rg[s].wait()
            l2c[s].wait()
            if s >= 2:
                oc[s - 2].wait()
            cur = s % 2
            rw_r, li_r, ob_r = rw[cur], li[cur], ob[cur]

            @plsc.parallel_loop(0, _T // _L, unroll=8)
            def _(k, rw_r=rw_r, li_r=li_r, ob_r=ob_r):
                idx16 = li_r[pl.ds(k * _L, _L)]
                r16 = lax.shift_right_logical(idx16, 4)
                c16 = lax.bitwise_and(idx16, 15)
                ob_r[pl.ds(k * _L, _L)] = plsc.load_gather(rw_r, [r16, c16])

            oc[s] = out_copy(s)
            if s + 2 < _BPW:
                rlc[s + 2] = rl_copy(s + 2)
                l2c[s + 2] = l2_copy(s + 2)
        oc[_BPW - 2].wait()
        oc[_BPW - 1].wait()

    p2 = pl.kernel(
        _p2_body,
        out_type=jax.ShapeDtypeStruct((_N,), jnp.float32),
        mesh=mesh,
        compiler_params=cparams,
        scratch_types=[
            pltpu.VMEM((r_pad,), jnp.int32),
            pltpu.VMEM((r_pad,), jnp.int32),
            pltpu.VMEM((r_pad, _L), jnp.float32),
            pltpu.VMEM((r_pad, _L), jnp.float32),
            pltpu.VMEM((_T,), jnp.int32),
            pltpu.VMEM((_T,), jnp.int32),
            pltpu.VMEM((_T,), jnp.float32),
            pltpu.VMEM((_T,), jnp.float32),
        ] + [pltpu.SemaphoreType.DMA] * 8,
    )

    z = p1(x, jnp.asarray(l1_np))
    out = p2(z, jnp.asarray(rl_np), jnp.asarray(l2_np))
    return out
